# baseline (speedup 1.0000x reference)
import functools

import jax
import jax.numpy as jnp
from jax import lax
from jax.experimental import pallas as pl
from jax.experimental.pallas import tpu as pltpu

F32 = jnp.float32
_INTERPRET = False
_CP = pltpu.CompilerParams(vmem_limit_bytes=120 * 1024 * 1024)


def _dot(a, b):
    return lax.dot_general(a, b, (((1,), (0,)), ((), ())),
                           preferred_element_type=F32)


def _tile(H, W, C, cap, mod8=False):
    best = 1 if not mod8 else min(8, H)
    for t in range(1, H + 1):
        if H % t == 0 and t * W * C <= cap and (not mod8 or t % 8 == 0
                                                or t == H):
            best = t
    return best


def _encmm_body(x_ref, w_ref, b_ref, m_ref, y_ref, s_ref, ss_ref):
    i = pl.program_id(0)
    y = _dot(x_ref[...], w_ref[...])
    y = (y + b_ref[...]) * m_ref[...]
    y_ref[...] = y

    @pl.when(i == 0)
    def _():
        s_ref[...] = jnp.zeros_like(s_ref)
        ss_ref[...] = jnp.zeros_like(ss_ref)

    s_ref[...] += jnp.sum(y, axis=0, keepdims=True)
    ss_ref[...] += jnp.sum(y * y, axis=0, keepdims=True)


def _enc_matmul(x, w, b, m):
    R, K = x.shape
    N = w.shape[1]
    Tr = _tile(R, K, 1, 160_000, mod8=True)
    y, s, ss = pl.pallas_call(
        _encmm_body,
        grid=(R // Tr,),
        in_specs=[pl.BlockSpec((Tr, K), lambda i: (i, 0)),
                  pl.BlockSpec((K, N), lambda i: (0, 0)),
                  pl.BlockSpec((1, N), lambda i: (0, 0)),
                  pl.BlockSpec((Tr, 1), lambda i: (i, 0))],
        out_specs=[pl.BlockSpec((Tr, N), lambda i: (i, 0)),
                   pl.BlockSpec((1, N), lambda i: (0, 0)),
                   pl.BlockSpec((1, N), lambda i: (0, 0))],
        out_shape=(jax.ShapeDtypeStruct((R, N), F32),
                   jax.ShapeDtypeStruct((1, N), F32),
                   jax.ShapeDtypeStruct((1, N), F32)),
        interpret=_INTERPRET,
        compiler_params=_CP,
    )(x, w, b.reshape(1, N), m)
    return y, s[0], ss[0]


def _pro_body(*refs, mode):
    o_ref = refs[-1]
    y = refs[0][...]
    if mode == 'fill':
        m = refs[1][...]
        a, bb, tok = refs[2][...][0], refs[3][...][0], refs[4][...][0]
        z = jnp.where(m > 0, y * a + bb, tok)
    elif mode == 'clip':
        a, bb = refs[1][...][0], refs[2][...][0]
        z = jnp.clip(y * a + bb, 0.0, 6.0)
    elif mode == 'affine':
        a, bb = refs[1][...][0], refs[2][...][0]
        z = y * a + bb
    elif mode == 'affine_skip':
        a, bb = refs[2][...][0], refs[3][...][0]
        z = y * a + bb + refs[1][...]
    o_ref[...] = z


def _pro(y, mode, *extras):
    B, H, W, C = y.shape
    Th = _tile(H, W, C, 200_000)
    ins = [y]
    in_specs = [pl.BlockSpec((1, Th, W, C), lambda b, i: (b, i, 0, 0))]
    for e in extras:
        if e.ndim == 4:
            ins.append(e)
            in_specs.append(pl.BlockSpec((1, Th, W, e.shape[3]),
                                         lambda b, i: (b, i, 0, 0)))
        else:
            ins.append(e.reshape(1, C))
            in_specs.append(pl.BlockSpec((1, C), lambda b, i: (0, 0)))
    return pl.pallas_call(
        functools.partial(_pro_body, mode=mode),
        grid=(B, H // Th),
        in_specs=in_specs,
        out_specs=pl.BlockSpec((1, Th, W, C), lambda b, i: (b, i, 0, 0)),
        out_shape=jax.ShapeDtypeStruct((B, H, W, C), F32),
        interpret=_INTERPRET,
        compiler_params=_CP,
    )(*ins)


def _pad(x):
    return jnp.pad(x, ((0, 0), (1, 1), (1, 1), (0, 0)))


def _conv3_body(x_ref, w_ref, *rest, W, C, N, Th, want_sums, has_bias,
                views):
    idx = 1 if has_bias else 0
    y_ref = rest[idx]
    b = pl.program_id(0)
    i = pl.program_id(1)
    row0 = i * Th
    acc = jnp.zeros((Th * W, N), F32)
    for dy in range(3):
        if views:
            rows = x_ref[0, dy]
        else:
            rows = x_ref[0, pl.ds(row0 + dy, Th), :, :]
        cat = jnp.concatenate([rows[:, dx:dx + W, :] for dx in range(3)],
                              axis=-1)
        acc = acc + _dot(cat.reshape(Th * W, 3 * C), w_ref[dy])
    if has_bias:
        acc = acc + rest[0][...]
    y_ref[...] = acc.reshape(1, Th, W, N)
    if want_sums:
        s_ref, ss_ref = rest[idx + 1], rest[idx + 2]

        @pl.when(jnp.logical_and(b == 0, i == 0))
        def _():
            s_ref[...] = jnp.zeros_like(s_ref)
            ss_ref[...] = jnp.zeros_like(ss_ref)

        s_ref[...] += jnp.sum(acc, axis=0, keepdims=True)
        ss_ref[...] += jnp.sum(acc * acc, axis=0, keepdims=True)


def _conv3(xp, w_oihw, bias=None, want_sums=False):
    B, Hp, Wp, C = xp.shape
    H, W = Hp - 2, Wp - 2
    N = w_oihw.shape[0]
    Th = _tile(H, W, 3 * C, 300_000)
    w = w_oihw.transpose(2, 3, 1, 0).reshape(3, 3 * C, N)
    views = Hp * Wp * max(C, 128) * 8 > 24_000_000
    if views:
        xs = jnp.stack([xp[:, d:d + H] for d in range(3)], axis=1)
        ins = [xs, w]
        in_specs = [pl.BlockSpec((1, 3, Th, Wp, C),
                                 lambda b, i: (b, 0, i, 0, 0)),
                    pl.BlockSpec((3, 3 * C, N), lambda b, i: (0, 0, 0))]
    else:
        ins = [xp, w]
        in_specs = [pl.BlockSpec((1, Hp, Wp, C), lambda b, i: (b, 0, 0, 0)),
                    pl.BlockSpec((3, 3 * C, N), lambda b, i: (0, 0, 0))]
    if bias is not None:
        ins.append(bias.reshape(1, N))
        in_specs.append(pl.BlockSpec((1, N), lambda b, i: (0, 0)))
    out_shape = [jax.ShapeDtypeStruct((B, H, W, N), F32)]
    out_specs = [pl.BlockSpec((1, Th, W, N), lambda b, i: (b, i, 0, 0))]
    if want_sums:
        out_shape += [jax.ShapeDtypeStruct((1, N), F32)] * 2
        out_specs += [pl.BlockSpec((1, N), lambda b, i: (0, 0))] * 2
    out = pl.pallas_call(
        functools.partial(_conv3_body, W=W, C=C, N=N, Th=Th,
                          want_sums=want_sums, has_bias=bias is not None,
                          views=views),
        grid=(B, H // Th),
        in_specs=in_specs,
        out_specs=out_specs,
        out_shape=out_shape,
        interpret=_INTERPRET,
        compiler_params=_CP,
    )(*ins)
    if want_sums:
        return out[0], out[1][0], out[2][0]
    return out[0]


def _convt_body(x_ref, w_ref, b_ref, *y_refs, W, C, Th):
    i = pl.program_id(1)
    row0 = i * Th
    bias = b_ref[...][0]
    for a in (0, 1):
        for bb in (0, 1):
            acc = jnp.zeros((Th * W, C), F32)
            for di in (0, 1):
                for dj in (0, 1):
                    win = x_ref[0, pl.ds(row0 + a + di, Th),
                                bb + dj:bb + dj + W, :]
                    acc = acc + _dot(win.reshape(Th * W, C),
                                     w_ref[a + 2 * di, bb + 2 * dj])
            y_refs[2 * a + bb][...] = (acc + bias).reshape(1, Th, W, C)


def _convt(xp, up_w, up_b):
    B, Hp, Wp, C = xp.shape
    H, W = Hp - 2, Wp - 2
    Th = _tile(H, W, C, 160_000)
    w = up_w.transpose(2, 3, 1, 0)
    outs = pl.pallas_call(
        functools.partial(_convt_body, W=W, C=C, Th=Th),
        grid=(B, H // Th),
        in_specs=[pl.BlockSpec((1, Hp, Wp, C), lambda b, i: (b, 0, 0, 0)),
                  pl.BlockSpec((4, 4, C, C), lambda b, i: (0, 0, 0, 0)),
                  pl.BlockSpec((1, C), lambda b, i: (0, 0))],
        out_specs=[pl.BlockSpec((1, Th, W, C),
                                lambda b, i: (b, i, 0, 0))] * 4,
        out_shape=[jax.ShapeDtypeStruct((B, H, W, C), F32)] * 4,
        interpret=_INTERPRET,
        compiler_params=_CP,
    )(xp, w, up_b.reshape(1, C))
    y = jnp.stack([jnp.stack(outs[0:2], axis=3),
                   jnp.stack(outs[2:4], axis=3)], axis=2)
    return y.reshape(B, 2 * H, 2 * W, C)


def _proj_body(x_ref, a_ref, b_ref, w_ref, bias_ref, o_ref):
    z = x_ref[0] * a_ref[...][0] + b_ref[...][0]
    o_ref[0] = _dot(z, w_ref[...]) + bias_ref[...]


def _proj(x, a, bshift, w_oihw, bias):
    B, H, W, C = x.shape
    N = w_oihw.shape[0]
    w = w_oihw.reshape(N, C).T
    R = H * W
    Tr = _tile(R, C, 1, 160_000, mod8=True)
    xr = x.reshape(B, R, C)
    y = pl.pallas_call(
        _proj_body,
        grid=(B, R // Tr),
        in_specs=[pl.BlockSpec((1, Tr, C), lambda b, i: (b, i, 0)),
                  pl.BlockSpec((1, C), lambda b, i: (0, 0)),
                  pl.BlockSpec((1, C), lambda b, i: (0, 0)),
                  pl.BlockSpec((C, N), lambda b, i: (0, 0)),
                  pl.BlockSpec((1, N), lambda b, i: (0, 0))],
        out_specs=pl.BlockSpec((1, Tr, N), lambda b, i: (b, i, 0)),
        out_shape=jax.ShapeDtypeStruct((B, R, N), F32),
        interpret=_INTERPRET,
        compiler_params=_CP,
    )(xr, a.reshape(1, C), bshift.reshape(1, C), w, bias.reshape(1, N))
    return y.reshape(B, H, W, N)


def _patchify(x, k):
    B, H, W, C = x.shape
    x = x.reshape(B, H // k, k, W // k, k, C).transpose(0, 1, 3, 2, 4, 5)
    return x.reshape(B * (H // k) * (W // k), k * k * C)


def _enc_weight(w_oihw):
    O, I, k, _ = w_oihw.shape
    return w_oihw.transpose(2, 3, 1, 0).reshape(k * k * I, O)


def _stats(s, ss, cnt, g, b, eps=1e-5):
    mean = s / cnt
    var = ss / cnt - mean * mean
    a = g * lax.rsqrt(var + eps)
    return a, b - mean * a


def kernel(inp_bchw, active_b1ff, params):
    B = inp_bchw.shape[0]
    act = active_b1ff[:, 0].astype(F32)
    count = jnp.sum(act)

    masks = {}
    for H in (56, 28, 14, 7):
        r = H // 7
        m = jnp.repeat(jnp.repeat(act, r, axis=1), r, axis=2)
        masks[H] = m[..., None]

    x = inp_bchw.transpose(0, 2, 3, 1)
    strides = [4, 2, 2, 2]
    res = [56, 28, 14, 7]
    feats, sums, sqs = [], [], []
    for i, p in enumerate(params['enc']):
        rows = _patchify(x, strides[i])
        H = res[i]
        y, s, ss = _enc_matmul(rows, _enc_weight(p['w']), p['b'],
                               masks[H].reshape(-1, 1))
        C = y.shape[1]
        x = y.reshape(B, H, H, C)
        feats.append(x)
        sums.append(s)
        sqs.append(ss)

    to_dec = []
    for (f, s, ss, p) in zip(feats[::-1], sums[::-1], sqs[::-1],
                             params['den']):
        H = f.shape[1]
        cnt = jnp.maximum(count * (H // 7) ** 2, 1.0)
        a, bsh = _stats(s, ss, cnt, p['bn_g'], p['bn_b'])
        z = _pro(f, 'fill', masks[H], a, bsh, p['mask_token'].reshape(-1))
        to_dec.append(_conv3(_pad(z), p['pw'], bias=p['pb']))

    y2, a2, b2 = None, None, None
    for i, p in enumerate(params['dec']):
        if i == 0:
            xp = _pad(to_dec[0])
        elif i < len(to_dec):
            xp = _pad(_pro(y2, 'affine_skip', to_dec[i], a2, b2))
        else:
            xp = _pad(_pro(y2, 'affine', a2, b2))
        up = _convt(xp, p['up_w'], p['up_b'])
        H2 = up.shape[1]
        y1, s1, ss1 = _conv3(_pad(up), p['c1_w'], want_sums=True)
        a1, b1 = _stats(s1, ss1, B * H2 * H2, p['bn1_g'], p['bn1_b'])
        z = _pro(y1, 'clip', a1, b1)
        y2, s2, ss2 = _conv3(_pad(z), p['c2_w'], want_sums=True)
        a2, b2 = _stats(s2, ss2, B * H2 * H2, p['bn2_g'], p['bn2_b'])

    rec = _proj(y2, a2, b2, params['proj_w'], params['proj_b'])
    return rec.transpose(0, 3, 1, 2)
